# baseline (speedup 1.0000x reference)
import functools

import jax
import jax.numpy as jnp
from jax import lax
from jax.experimental import pallas as pl
from jax.experimental.pallas import tpu as pltpu
from jax.experimental.pallas import tpu_sc as plsc

D = 256
K = 8192
N = 8192
NB = 8
KB = 8
BN = N // NB
BK = K // KB

_SC_NC = 2
_SC_NS = 16
_SC_NW = _SC_NC * _SC_NS
_B_PER_W = N // _SC_NW


def _fused_body(xt_ref, w2_ref, idx_ref, enc_ref, loss_ref, perp_ref,
                mv_s, mi_s, mi_prev, s1_s, cnt_s, acc_s):
    nb = pl.program_id(0)
    kb = pl.program_id(1)

    @pl.when(nb > 0)
    def _():
        idx_col = lax.transpose(mi_prev[...], (1, 0))
        ids_n = lax.broadcasted_iota(jnp.int32, (BN, BK), 1) + kb * BK
        onehot = (ids_n == idx_col).astype(jnp.float32)
        enc_ref[...] = onehot
        ones = jnp.ones((1, BN), jnp.float32)
        cnt = lax.dot_general(ones, onehot, (((1,), (0,)), ((), ())),
                              preferred_element_type=jnp.float32)

        @pl.when(nb == 1)
        def _():
            cnt_s[:, pl.ds(kb * BK, BK)] = cnt

        @pl.when(nb > 1)
        def _():
            cnt_s[:, pl.ds(kb * BK, BK)] = cnt_s[:, pl.ds(kb * BK, BK)] + cnt

    @pl.when(nb < NB)
    def _():
        xt = xt_ref[...]
        w2 = w2_ref[...]
        s = lax.dot_general(w2, xt, (((1,), (0,)), ((), ())),
                            preferred_element_type=jnp.float32)

        @pl.when(kb == 0)
        def _():
            s1_s[...] = jnp.sum(xt * xt, axis=0, keepdims=True)

        s2 = 0.25 * jnp.sum(w2 * w2, axis=1, keepdims=True)
        d = (s1_s[...] + s) + s2
        bmin = jnp.min(d, axis=0, keepdims=True)
        ids = lax.broadcasted_iota(jnp.int32, (BK, BN), 0) + kb * BK
        bidx = jnp.min(jnp.where(d == bmin, ids, jnp.int32(2**30)),
                       axis=0, keepdims=True)

        @pl.when(kb == 0)
        def _():
            mv_s[...] = bmin
            mi_s[...] = bidx

        @pl.when(kb > 0)
        def _():
            better = bmin < mv_s[...]
            mi_s[...] = jnp.where(better, bidx, mi_s[...])
            mv_s[...] = jnp.where(better, bmin, mv_s[...])

        @pl.when(kb == KB - 1)
        def _():
            idx_ref[...] = mi_s[...].reshape(1, 1, BN)
            mi_prev[...] = mi_s[...]
            rowsum = jnp.sum(mv_s[...])

            @pl.when(nb == 0)
            def _():
                acc_s[0, 0] = rowsum

            @pl.when(nb > 0)
            def _():
                acc_s[0, 0] = acc_s[0, 0] + rowsum

    @pl.when(jnp.logical_and(nb == NB, kb == KB - 1))
    def _():
        loss = 0.25 * acc_s[0, 0] * (1.0 / (N * D))
        loss_ref[...] = jnp.full((1, 128), loss, jnp.float32)
        p = cnt_s[...] * (1.0 / N)
        ent = jnp.sum(p * jnp.log(p + 1e-10))
        perp_ref[...] = jnp.exp(jnp.full((1, 128), -ent, jnp.float32))


def _sc_gather(table_hbm, idx_hbm, out_hbm, idx_v, rows_v, sem):
    wid = lax.axis_index("s") * _SC_NC + lax.axis_index("c")
    base = wid * _B_PER_W
    pltpu.sync_copy(idx_hbm.at[pl.ds(base, _B_PER_W)], idx_v)
    pltpu.async_copy(table_hbm.at[idx_v], rows_v, sem).wait()
    pltpu.sync_copy(rows_v, out_hbm.at[pl.ds(base, _B_PER_W)])


def kernel(inputTensor, emb_weights, ema_w, ema_cluster_size):
    del ema_w, ema_cluster_size
    flat = inputTensor.reshape(-1, D)
    xt = flat.T
    w2 = -2.0 * emb_weights

    idx3, enc, loss_out, perp_out = pl.pallas_call(
        _fused_body,
        grid=(NB + 1, KB),
        in_specs=[
            pl.BlockSpec((D, BN), lambda nb, kb: (0, jnp.minimum(nb, NB - 1))),
            pl.BlockSpec((BK, D), lambda nb, kb: (kb, 0)),
        ],
        out_specs=[
            pl.BlockSpec((1, 1, BN),
                         lambda nb, kb: (jnp.minimum(nb, NB - 1), 0, 0)),
            pl.BlockSpec((BN, BK),
                         lambda nb, kb: (jnp.maximum(nb - 1, 0), kb)),
            pl.BlockSpec((1, 128), lambda nb, kb: (0, 0)),
            pl.BlockSpec((1, 128), lambda nb, kb: (0, 0)),
        ],
        out_shape=[
            jax.ShapeDtypeStruct((NB, 1, BN), jnp.int32),
            jax.ShapeDtypeStruct((N, K), jnp.float32),
            jax.ShapeDtypeStruct((1, 128), jnp.float32),
            jax.ShapeDtypeStruct((1, 128), jnp.float32),
        ],
        scratch_shapes=[
            pltpu.VMEM((1, BN), jnp.float32),
            pltpu.VMEM((1, BN), jnp.int32),
            pltpu.VMEM((1, BN), jnp.int32),
            pltpu.VMEM((1, BN), jnp.float32),
            pltpu.VMEM((1, K), jnp.float32),
            pltpu.SMEM((1, 1), jnp.float32),
        ],
    )(xt, w2)

    idx_flat = idx3.reshape(N)

    sc_gather = functools.partial(
        pl.kernel,
        mesh=plsc.VectorSubcoreMesh(core_axis_name="c", subcore_axis_name="s"),
        out_type=jax.ShapeDtypeStruct((N, D), jnp.float32),
        scratch_types=[
            pltpu.VMEM((_B_PER_W,), jnp.int32),
            pltpu.VMEM((_B_PER_W, D), jnp.float32),
            pltpu.SemaphoreType.DMA,
        ],
    )(_sc_gather)
    quantized = sc_gather(emb_weights, idx_flat)

    loss = loss_out[0, 0]
    perplexity = perp_out[0, 0]
    quantized_st = quantized.reshape(inputTensor.shape)
    return (loss, quantized_st, perplexity, enc)
